# Initial kernel scaffold; baseline (speedup 1.0000x reference)
import jax
import jax.numpy as jnp
from jax.experimental import pallas as pl


def kernel(edge_index, surface_batch, part_batch, edge_weight):
    raise NotImplementedError("write your pallas kernel here")


# speedup vs baseline: 401.5368x; 401.5368x over previous
import functools

import jax
import jax.numpy as jnp
from jax import lax
from jax.experimental import pallas as pl
from jax.experimental.pallas import tpu as pltpu
from jax.experimental.pallas import tpu_sc as plsc

N_NODES = 100000
N_EDGES = 3200000
N_GRAPHS = 256
N_PARTS = 2048

NC = 2
NS = 16
NW = NC * NS
LANES = 16

E_PER_W = N_EDGES // NW
CHUNK = 4000
N_CHUNKS = E_PER_W // CHUNK
ITERS = CHUNK // LANES


def _table_body(pb_ref, tbl_ref):
    pb = pb_ref[...].reshape(1, N_PARTS)
    gids = lax.broadcasted_iota(jnp.int32, (N_GRAPHS, N_PARTS), 0)
    counts = jnp.sum((pb == gids).astype(jnp.float32), axis=1)
    tbl_ref[...] = lax.rsqrt(counts)


def _make_table(part_batch):
    return pl.pallas_call(
        _table_body,
        out_shape=jax.ShapeDtypeStruct((N_GRAPHS,), jnp.float32),
    )(part_batch)


_MESH = plsc.VectorSubcoreMesh(core_axis_name="c", subcore_axis_name="s")


@functools.partial(
    pl.kernel,
    out_type=jax.ShapeDtypeStruct((N_EDGES,), jnp.float32),
    mesh=_MESH,
    compiler_params=pltpu.CompilerParams(needs_layout_passes=False),
    scratch_types=[
        pltpu.VMEM((N_NODES,), jnp.int32),
        pltpu.VMEM((N_GRAPHS,), jnp.float32),
        pltpu.VMEM((CHUNK,), jnp.int32),
        pltpu.VMEM((CHUNK,), jnp.float32),
        pltpu.VMEM((CHUNK,), jnp.float32),
    ],
)
def _edge_kernel(src_hbm, w_hbm, sb_hbm, tbl_hbm, out_hbm,
                 sb_v, tbl_v, idx_v, w_v, o_v):
    wid = lax.axis_index("s") * NC + lax.axis_index("c")
    pltpu.sync_copy(sb_hbm, sb_v)
    pltpu.sync_copy(tbl_hbm, tbl_v)
    base_w = wid * E_PER_W

    def chunk_body(c, _):
        base = base_w + c * CHUNK
        pltpu.sync_copy(src_hbm.at[pl.ds(base, CHUNK)], idx_v)
        pltpu.sync_copy(w_hbm.at[pl.ds(base, CHUNK)], w_v)

        def body(i, _):
            s = pl.ds(i * LANES, LANES)
            gi = plsc.load_gather(sb_v, [idx_v[s]])
            v = plsc.load_gather(tbl_v, [gi])
            o_v[s] = v * w_v[s]
            return 0

        lax.fori_loop(0, ITERS, body, 0)
        pltpu.sync_copy(o_v, out_hbm.at[pl.ds(base, CHUNK)])
        return 0

    lax.fori_loop(0, N_CHUNKS, chunk_body, 0)


def kernel(edge_index, surface_batch, part_batch, edge_weight):
    tbl = _make_table(part_batch)
    src = edge_index[0]
    return _edge_kernel(src, edge_weight, surface_batch, tbl)


# speedup vs baseline: 593.4666x; 1.4780x over previous
import functools

import jax
import jax.numpy as jnp
from jax import lax
from jax.experimental import pallas as pl
from jax.experimental.pallas import tpu as pltpu
from jax.experimental.pallas import tpu_sc as plsc

N_NODES = 100000
N_EDGES = 3200000
N_GRAPHS = 256
N_PARTS = 2048

NC = 2
NS = 16
NW = NC * NS
LANES = 16

E_PER_W = N_EDGES // NW
CHUNK = 4000
N_CHUNKS = E_PER_W // CHUNK
ITERS = CHUNK // LANES


def _table_body(pb_ref, tbl_ref):
    pb = pb_ref[...].reshape(1, N_PARTS)
    gids = lax.broadcasted_iota(jnp.int32, (N_GRAPHS, N_PARTS), 0)
    counts = jnp.sum((pb == gids).astype(jnp.float32), axis=1)
    tbl_ref[...] = lax.rsqrt(counts)


def _make_table(part_batch):
    return pl.pallas_call(
        _table_body,
        out_shape=jax.ShapeDtypeStruct((N_GRAPHS,), jnp.float32),
    )(part_batch)


_MESH = plsc.VectorSubcoreMesh(core_axis_name="c", subcore_axis_name="s")


@functools.partial(
    pl.kernel,
    out_type=jax.ShapeDtypeStruct((N_EDGES,), jnp.float32),
    mesh=_MESH,
    compiler_params=pltpu.CompilerParams(needs_layout_passes=False),
    scratch_types=[
        pltpu.VMEM((N_NODES,), jnp.int32),
        pltpu.VMEM((N_GRAPHS,), jnp.float32),
        pltpu.VMEM((CHUNK,), jnp.int32),
        pltpu.VMEM((CHUNK,), jnp.float32),
        pltpu.VMEM((CHUNK,), jnp.float32),
    ],
)
def _edge_kernel(src_hbm, w_hbm, sb_hbm, tbl_hbm, out_hbm,
                 sb_v, tbl_v, idx_v, w_v, o_v):
    wid = lax.axis_index("s") * NC + lax.axis_index("c")
    pltpu.sync_copy(sb_hbm, sb_v)
    pltpu.sync_copy(tbl_hbm, tbl_v)
    base_w = wid * E_PER_W

    def chunk_body(c, _):
        base = base_w + c * CHUNK
        pltpu.sync_copy(src_hbm.at[pl.ds(base, CHUNK)], idx_v)
        pltpu.sync_copy(w_hbm.at[pl.ds(base, CHUNK)], w_v)

        @plsc.parallel_loop(0, CHUNK, step=LANES, unroll=8)
        def _(e):
            s = pl.ds(e, LANES)
            gi = plsc.load_gather(sb_v, [idx_v[s]])
            v = plsc.load_gather(tbl_v, [gi])
            o_v[s] = v * w_v[s]
        pltpu.sync_copy(o_v, out_hbm.at[pl.ds(base, CHUNK)])
        return 0

    lax.fori_loop(0, N_CHUNKS, chunk_body, 0)


def kernel(edge_index, surface_batch, part_batch, edge_weight):
    tbl = _make_table(part_batch)
    src = edge_index[0]
    return _edge_kernel(src, edge_weight, surface_batch, tbl)


# speedup vs baseline: 807.7058x; 1.3610x over previous
import functools

import jax
import jax.numpy as jnp
from jax import lax
from jax.experimental import pallas as pl
from jax.experimental.pallas import tpu as pltpu
from jax.experimental.pallas import tpu_sc as plsc

N_NODES = 100000
N_EDGES = 3200000
N_GRAPHS = 256
N_PARTS = 2048

NC = 2
NS = 16
NW = NC * NS
LANES = 16

E_PER_W = N_EDGES // NW
CHUNK = 2000
N_CHUNKS = E_PER_W // CHUNK
N_PAIRS = N_CHUNKS // 2


def _table_body(pb_ref, tbl_ref):
    pb = pb_ref[...].reshape(1, N_PARTS)
    gids = lax.broadcasted_iota(jnp.int32, (N_GRAPHS, N_PARTS), 0)
    counts = jnp.sum((pb == gids).astype(jnp.float32), axis=1)
    tbl_ref[...] = lax.rsqrt(counts)


def _make_table(part_batch):
    return pl.pallas_call(
        _table_body,
        out_shape=jax.ShapeDtypeStruct((N_GRAPHS,), jnp.float32),
    )(part_batch)


_MESH = plsc.VectorSubcoreMesh(core_axis_name="c", subcore_axis_name="s")


@functools.partial(
    pl.kernel,
    out_type=jax.ShapeDtypeStruct((N_EDGES,), jnp.float32),
    mesh=_MESH,
    compiler_params=pltpu.CompilerParams(needs_layout_passes=False),
    scratch_types=[
        pltpu.VMEM((N_NODES,), jnp.int32),
        pltpu.VMEM((N_GRAPHS,), jnp.float32),
        pltpu.VMEM((CHUNK,), jnp.int32),
        pltpu.VMEM((CHUNK,), jnp.int32),
        pltpu.VMEM((CHUNK,), jnp.float32),
        pltpu.VMEM((CHUNK,), jnp.float32),
        pltpu.VMEM((CHUNK,), jnp.float32),
        pltpu.VMEM((CHUNK,), jnp.float32),
        pltpu.SemaphoreType.DMA,
        pltpu.SemaphoreType.DMA,
        pltpu.SemaphoreType.DMA,
        pltpu.SemaphoreType.DMA,
    ],
)
def _edge_kernel(src_hbm, w_hbm, sb_hbm, tbl_hbm, out_hbm,
                 sb_v, tbl_v, idx0, idx1, w0, w1, o0, o1,
                 isem0, isem1, osem0, osem1):
    wid = lax.axis_index("s") * NC + lax.axis_index("c")
    pltpu.sync_copy(sb_hbm, sb_v)
    pltpu.sync_copy(tbl_hbm, tbl_v)
    base_w = wid * E_PER_W

    bufs = ((idx0, w0, o0, isem0, osem0), (idx1, w1, o1, isem1, osem1))

    def issue_in(c, idx_v, w_v, isem):
        base = base_w + c * CHUNK
        pltpu.async_copy(src_hbm.at[pl.ds(base, CHUNK)], idx_v, isem)
        pltpu.async_copy(w_hbm.at[pl.ds(base, CHUNK)], w_v, isem)

    def wait_in(idx_v, w_v, isem):
        pltpu.make_async_copy(src_hbm.at[pl.ds(0, CHUNK)], idx_v, isem).wait()
        pltpu.make_async_copy(w_hbm.at[pl.ds(0, CHUNK)], w_v, isem).wait()

    def wait_out(o_v, osem):
        pltpu.make_async_copy(o_v, out_hbm.at[pl.ds(0, CHUNK)], osem).wait()

    issue_in(0, idx0, w0, isem0)
    issue_in(1, idx1, w1, isem1)

    def pair_body(p, _):
        for b in (0, 1):
            idx_v, w_v, o_v, isem, osem = bufs[b]
            c = p * 2 + b
            wait_in(idx_v, w_v, isem)

            @pl.when(p >= 1)
            def _():
                wait_out(o_v, osem)

            @plsc.parallel_loop(0, CHUNK, step=LANES, unroll=8)
            def _(e):
                s = pl.ds(e, LANES)
                gi = plsc.load_gather(sb_v, [idx_v[s]])
                v = plsc.load_gather(tbl_v, [gi])
                o_v[s] = v * w_v[s]

            pltpu.async_copy(o_v, out_hbm.at[pl.ds(base_w + c * CHUNK, CHUNK)],
                             osem)

            @pl.when(p < N_PAIRS - 1)
            def _():
                issue_in(c + 2, idx_v, w_v, isem)
        return 0

    lax.fori_loop(0, N_PAIRS, pair_body, 0)
    wait_out(o0, osem0)
    wait_out(o1, osem1)


def kernel(edge_index, surface_batch, part_batch, edge_weight):
    tbl = _make_table(part_batch)
    src = edge_index[0]
    return _edge_kernel(src, edge_weight, surface_batch, tbl)


# speedup vs baseline: 828.3545x; 1.0256x over previous
import functools

import jax
import jax.numpy as jnp
from jax import lax
from jax.experimental import pallas as pl
from jax.experimental.pallas import tpu as pltpu
from jax.experimental.pallas import tpu_sc as plsc

N_NODES = 100000
N_EDGES = 3200000
N_GRAPHS = 256
N_PARTS = 2048

NC = 2
NS = 16
NW = NC * NS
LANES = 16

E_PER_W = N_EDGES // NW
CHUNK = 2000
N_CHUNKS = E_PER_W // CHUNK
N_PAIRS = N_CHUNKS // 2


_MESH = plsc.VectorSubcoreMesh(core_axis_name="c", subcore_axis_name="s")


def _build_table(pb_v, start_v, end_v, tbl_v):
    lane = lax.broadcasted_iota(jnp.int32, (LANES,), 0)
    for g in range(N_GRAPHS // LANES):
        s = pl.ds(g * LANES, LANES)
        start_v[s] = jnp.zeros((LANES,), jnp.int32)
        end_v[s] = jnp.full((LANES,), -1, jnp.int32)

    def scan_body(i, _):
        e_vec = i * LANES + lane
        cur = pb_v[pl.ds(i * LANES, LANES)]
        prev = plsc.load_gather(pb_v, [jnp.maximum(e_vec - 1, 0)])
        nxt = plsc.load_gather(pb_v, [jnp.minimum(e_vec + 1, N_PARTS - 1)])
        plsc.store_scatter(start_v, [cur], e_vec,
                           mask=(cur != prev) | (e_vec == 0))
        plsc.store_scatter(end_v, [cur], e_vec,
                           mask=(cur != nxt) | (e_vec == N_PARTS - 1))
        return 0

    lax.fori_loop(0, N_PARTS // LANES, scan_body, 0)

    for g in range(N_GRAPHS // LANES):
        s = pl.ds(g * LANES, LANES)
        d = (end_v[s] - start_v[s] + 1).astype(jnp.float32)
        i32v = plsc.bitcast(d, jnp.int32)
        y = plsc.bitcast(jnp.int32(0x5F3759DF) - (i32v >> 1), jnp.float32)
        hd = 0.5 * d
        y = y * (1.5 - hd * y * y)
        y = y * (1.5 - hd * y * y)
        y = y * (1.5 - hd * y * y)
        tbl_v[s] = jnp.where(d == 0.0, jnp.float32(jnp.inf), y)


@functools.partial(
    pl.kernel,
    out_type=jax.ShapeDtypeStruct((N_EDGES,), jnp.float32),
    mesh=_MESH,
    compiler_params=pltpu.CompilerParams(needs_layout_passes=False),
    scratch_types=[
        pltpu.VMEM((N_NODES,), jnp.int32),
        pltpu.VMEM((N_GRAPHS,), jnp.float32),
        pltpu.VMEM((N_PARTS,), jnp.int32),
        pltpu.VMEM((N_GRAPHS,), jnp.int32),
        pltpu.VMEM((N_GRAPHS,), jnp.int32),
        pltpu.VMEM((CHUNK,), jnp.int32),
        pltpu.VMEM((CHUNK,), jnp.int32),
        pltpu.VMEM((CHUNK,), jnp.float32),
        pltpu.VMEM((CHUNK,), jnp.float32),
        pltpu.VMEM((CHUNK,), jnp.float32),
        pltpu.VMEM((CHUNK,), jnp.float32),
        pltpu.SemaphoreType.DMA,
        pltpu.SemaphoreType.DMA,
        pltpu.SemaphoreType.DMA,
        pltpu.SemaphoreType.DMA,
    ],
)
def _edge_kernel(src_hbm, w_hbm, sb_hbm, pb_hbm, out_hbm,
                 sb_v, tbl_v, pb_v, start_v, end_v, idx0, idx1, w0, w1, o0, o1,
                 isem0, isem1, osem0, osem1):
    wid = lax.axis_index("s") * NC + lax.axis_index("c")
    pltpu.sync_copy(pb_hbm, pb_v)
    pltpu.sync_copy(sb_hbm, sb_v)
    _build_table(pb_v, start_v, end_v, tbl_v)
    base_w = wid * E_PER_W

    bufs = ((idx0, w0, o0, isem0, osem0), (idx1, w1, o1, isem1, osem1))

    def issue_in(c, idx_v, w_v, isem):
        base = base_w + c * CHUNK
        pltpu.async_copy(src_hbm.at[pl.ds(base, CHUNK)], idx_v, isem)
        pltpu.async_copy(w_hbm.at[pl.ds(base, CHUNK)], w_v, isem)

    def wait_in(idx_v, w_v, isem):
        pltpu.make_async_copy(src_hbm.at[pl.ds(0, CHUNK)], idx_v, isem).wait()
        pltpu.make_async_copy(w_hbm.at[pl.ds(0, CHUNK)], w_v, isem).wait()

    def wait_out(o_v, osem):
        pltpu.make_async_copy(o_v, out_hbm.at[pl.ds(0, CHUNK)], osem).wait()

    issue_in(0, idx0, w0, isem0)
    issue_in(1, idx1, w1, isem1)

    def pair_body(p, _):
        for b in (0, 1):
            idx_v, w_v, o_v, isem, osem = bufs[b]
            c = p * 2 + b
            wait_in(idx_v, w_v, isem)

            @pl.when(p >= 1)
            def _():
                wait_out(o_v, osem)

            @plsc.parallel_loop(0, CHUNK, step=LANES, unroll=8)
            def _(e):
                s = pl.ds(e, LANES)
                gi = plsc.load_gather(sb_v, [idx_v[s]])
                v = plsc.load_gather(tbl_v, [gi])
                o_v[s] = v * w_v[s]

            pltpu.async_copy(o_v, out_hbm.at[pl.ds(base_w + c * CHUNK, CHUNK)],
                             osem)

            @pl.when(p < N_PAIRS - 1)
            def _():
                issue_in(c + 2, idx_v, w_v, isem)
        return 0

    lax.fori_loop(0, N_PAIRS, pair_body, 0)
    wait_out(o0, osem0)
    wait_out(o1, osem1)


def kernel(edge_index, surface_batch, part_batch, edge_weight):
    src = edge_index[0]
    return _edge_kernel(src, edge_weight, surface_batch, part_batch)
